# baseline (speedup 1.0000x reference)
import jax
import jax.numpy as jnp
from jax import lax
from jax.experimental import pallas as pl
from jax.experimental.pallas import tpu as pltpu
from jax.experimental.pallas import tpu_sc as plsc

_NC, _NS, _L = 2, 16, 16
_NW = _NC * _NS
_CH = 8
_B, _S, _D = 4, 2048, 4096


def _sc_body(x_hbm, tab_hbm, o_hbm, xb0, xb1, tbuf, ls0, ls1, ss0, ss1):
    rows_per_w = _S // _NW
    n_steps = (rows_per_w // _CH) * _B
    wid = lax.axis_index("c") * _NS + lax.axis_index("s")
    s_base = wid * rows_per_w

    xbufs = (xb0, xb1)
    lsems = (ls0, ls1)
    ssems = (ss0, ss1)

    def src_slice(t):
        c, b = t // _B, t % _B
        return (b, pl.ds(s_base + c * _CH, _CH))

    def load(t):
        b, sl = src_slice(t)
        return pltpu.make_async_copy(x_hbm.at[b, sl], xbufs[t % 2], lsems[t % 2])

    def store(t):
        b, sl = src_slice(t)
        return pltpu.make_async_copy(xbufs[t % 2], o_hbm.at[b, sl], ssems[t % 2])

    load(0).start()
    for t in range(n_steps):
        if t + 1 < n_steps:
            if t >= 1:
                store(t - 1).wait()
            load(t + 1).start()
        if t % _B == 0:
            pltpu.sync_copy(
                tab_hbm.at[pl.ds(s_base + (t // _B) * _CH, _CH)], tbuf)
        load(t).wait()

        xb = xbufs[t % 2]

        def row_body(r, _):
            def add_body(g, _):
                plsc.addupdate(xb.at[r, pl.ds(g * _L, _L)],
                               tbuf[r, pl.ds(g * _L, _L)])
                return 0

            return lax.fori_loop(0, _D // _L, add_body, 0, unroll=16)

        lax.fori_loop(0, _CH, row_body, 0)
        store(t).start()
    store(n_steps - 2).wait()
    store(n_steps - 1).wait()


def kernel(x, pos_table):
    B, S, D = x.shape
    return pl.kernel(
        _sc_body,
        out_type=jax.ShapeDtypeStruct((B, S, D), x.dtype),
        mesh=plsc.VectorSubcoreMesh(core_axis_name="c", subcore_axis_name="s"),
        scratch_types=[
            pltpu.VMEM((_CH, _D), jnp.float32),
            pltpu.VMEM((_CH, _D), jnp.float32),
            pltpu.VMEM((_CH, _D), jnp.float32),
            pltpu.SemaphoreType.DMA,
            pltpu.SemaphoreType.DMA,
            pltpu.SemaphoreType.DMA,
            pltpu.SemaphoreType.DMA,
        ],
    )(x, pos_table)


# speedup vs baseline: 1.0164x; 1.0164x over previous
import jax
import jax.numpy as jnp
from jax import lax
from jax.experimental import pallas as pl
from jax.experimental.pallas import tpu as pltpu
from jax.experimental.pallas import tpu_sc as plsc

_NC, _NS, _L = 2, 16, 16
_NW = _NC * _NS
_CH = 8
_B, _S, _D = 4, 2048, 4096


def _sc_body(x_hbm, tab_hbm, o_hbm, xb0, xb1, tbuf, ls0, ls1, ss0, ss1):
    rows_per_w = _S // _NW
    n_steps = (rows_per_w // _CH) * _B
    wid = lax.axis_index("c") * _NS + lax.axis_index("s")
    s_base = wid * rows_per_w

    xbufs = (xb0, xb1)
    lsems = (ls0, ls1)
    ssems = (ss0, ss1)

    def src_slice(t):
        c, b = t // _B, t % _B
        return (b, pl.ds(s_base + c * _CH, _CH))

    def load(t):
        b, sl = src_slice(t)
        return pltpu.make_async_copy(x_hbm.at[b, sl], xbufs[t % 2], lsems[t % 2])

    def store(t):
        b, sl = src_slice(t)
        return pltpu.make_async_copy(xbufs[t % 2], o_hbm.at[b, sl], ssems[t % 2])

    load(0).start()
    for t in range(n_steps):
        if t + 1 < n_steps:
            if t >= 1:
                store(t - 1).wait()
            load(t + 1).start()
        if t % _B == 0:
            pltpu.sync_copy(
                tab_hbm.at[pl.ds(s_base + (t // _B) * _CH, _CH)], tbuf)
        load(t).wait()

        xb = xbufs[t % 2]

        def row_body(r, _):
            @plsc.parallel_loop(0, _D, step=_L, unroll=8)
            def add_body(g):
                plsc.addupdate(xb.at[r, pl.ds(g, _L)], tbuf[r, pl.ds(g, _L)])

            return 0

        lax.fori_loop(0, _CH, row_body, 0)
        store(t).start()
    store(n_steps - 2).wait()
    store(n_steps - 1).wait()


def kernel(x, pos_table):
    B, S, D = x.shape
    return pl.kernel(
        _sc_body,
        out_type=jax.ShapeDtypeStruct((B, S, D), x.dtype),
        mesh=plsc.VectorSubcoreMesh(core_axis_name="c", subcore_axis_name="s"),
        scratch_types=[
            pltpu.VMEM((_CH, _D), jnp.float32),
            pltpu.VMEM((_CH, _D), jnp.float32),
            pltpu.VMEM((_CH, _D), jnp.float32),
            pltpu.SemaphoreType.DMA,
            pltpu.SemaphoreType.DMA,
            pltpu.SemaphoreType.DMA,
            pltpu.SemaphoreType.DMA,
        ],
    )(x, pos_table)


# speedup vs baseline: 1.1469x; 1.1285x over previous
import jax
import jax.numpy as jnp
from jax import lax
from jax.experimental import pallas as pl
from jax.experimental.pallas import tpu as pltpu
from jax.experimental.pallas import tpu_sc as plsc

_NC, _NS, _L = 2, 16, 16
_NW = _NC * _NS
_CH = 4
_B, _S, _D = 4, 2048, 4096


def _sc_body(x_hbm, tab_hbm, o_hbm,
             xb0, xb1, xb2, tb0, tb1,
             ls0, ls1, ls2, ss0, ss1, ss2, ts0, ts1):
    rows_per_w = _S // _NW
    n_chunks = rows_per_w // _CH
    n_steps = n_chunks * _B
    wid = lax.axis_index("c") * _NS + lax.axis_index("s")
    s_base = wid * rows_per_w

    xbufs = (xb0, xb1, xb2)
    lsems = (ls0, ls1, ls2)
    ssems = (ss0, ss1, ss2)
    tbufs = (tb0, tb1)
    tsems = (ts0, ts1)

    def src_slice(t):
        c, b = t // _B, t % _B
        return (b, pl.ds(s_base + c * _CH, _CH))

    def load(t):
        b, sl = src_slice(t)
        return pltpu.make_async_copy(x_hbm.at[b, sl], xbufs[t % 3], lsems[t % 3])

    def store(t):
        b, sl = src_slice(t)
        return pltpu.make_async_copy(xbufs[t % 3], o_hbm.at[b, sl], ssems[t % 3])

    def tload(c):
        return pltpu.make_async_copy(
            tab_hbm.at[pl.ds(s_base + c * _CH, _CH)], tbufs[c % 2], tsems[c % 2])

    tload(0).start()
    load(0).start()
    load(1).start()
    for t in range(n_steps):
        c = t // _B
        if t + 2 < n_steps:
            if t >= 1:
                store(t - 1).wait()
            load(t + 2).start()
        if t % _B == 0:
            if c + 1 < n_chunks:
                tload(c + 1).start()
            tload(c).wait()
        load(t).wait()

        xb, tb = xbufs[t % 3], tbufs[c % 2]

        def row_body(r, _):
            @plsc.parallel_loop(0, _D, step=_L, unroll=8)
            def add_body(g):
                plsc.addupdate(xb.at[r, pl.ds(g, _L)], tb[r, pl.ds(g, _L)])

            return 0

        lax.fori_loop(0, _CH, row_body, 0)
        store(t).start()
    store(n_steps - 3).wait()
    store(n_steps - 2).wait()
    store(n_steps - 1).wait()


def kernel(x, pos_table):
    B, S, D = x.shape
    return pl.kernel(
        _sc_body,
        out_type=jax.ShapeDtypeStruct((B, S, D), x.dtype),
        mesh=plsc.VectorSubcoreMesh(core_axis_name="c", subcore_axis_name="s"),
        scratch_types=(
            [pltpu.VMEM((_CH, _D), jnp.float32)] * 5
            + [pltpu.SemaphoreType.DMA] * 8
        ),
    )(x, pos_table)


# speedup vs baseline: 1.1517x; 1.0041x over previous
import jax
import jax.numpy as jnp
from jax import lax
from jax.experimental import pallas as pl
from jax.experimental.pallas import tpu as pltpu
from jax.experimental.pallas import tpu_sc as plsc

_NC, _NS, _L = 2, 16, 16
_NW = _NC * _NS
_CH = 4
_B, _S, _D = 4, 2048, 4096


def _sc_body(x_hbm, tab_hbm, o_hbm,
             xb0, xb1, xb2, xb3, tb0, tb1,
             ls0, ls1, ls2, ls3, ss0, ss1, ss2, ss3, ts0, ts1):
    rows_per_w = _S // _NW
    n_chunks = rows_per_w // _CH
    n_steps = n_chunks * _B
    wid = lax.axis_index("c") * _NS + lax.axis_index("s")
    s_base = wid * rows_per_w

    xbufs = (xb0, xb1, xb2, xb3)
    lsems = (ls0, ls1, ls2, ls3)
    ssems = (ss0, ss1, ss2, ss3)
    tbufs = (tb0, tb1)
    tsems = (ts0, ts1)

    def src_slice(t):
        c, b = t // _B, t % _B
        return (b, pl.ds(s_base + c * _CH, _CH))

    def load(t):
        b, sl = src_slice(t)
        return pltpu.make_async_copy(x_hbm.at[b, sl], xbufs[t % 4], lsems[t % 4])

    def store(t):
        b, sl = src_slice(t)
        return pltpu.make_async_copy(xbufs[t % 4], o_hbm.at[b, sl], ssems[t % 4])

    def tload(c):
        return pltpu.make_async_copy(
            tab_hbm.at[pl.ds(s_base + c * _CH, _CH)], tbufs[c % 2], tsems[c % 2])

    tload(0).start()
    load(0).start()
    load(1).start()
    load(2).start()
    for t in range(n_steps):
        c = t // _B
        if t + 3 < n_steps:
            if t >= 1:
                store(t - 1).wait()
            load(t + 3).start()
        if t % _B == 0:
            if c + 1 < n_chunks:
                tload(c + 1).start()
            tload(c).wait()
        load(t).wait()

        xb, tb = xbufs[t % 4], tbufs[c % 2]

        def row_body(r, _):
            @plsc.parallel_loop(0, _D, step=_L, unroll=8)
            def add_body(g):
                plsc.addupdate(xb.at[r, pl.ds(g, _L)], tb[r, pl.ds(g, _L)])

            return 0

        lax.fori_loop(0, _CH, row_body, 0)
        store(t).start()
    store(n_steps - 4).wait()
    store(n_steps - 3).wait()
    store(n_steps - 2).wait()
    store(n_steps - 1).wait()


def kernel(x, pos_table):
    B, S, D = x.shape
    return pl.kernel(
        _sc_body,
        out_type=jax.ShapeDtypeStruct((B, S, D), x.dtype),
        mesh=plsc.VectorSubcoreMesh(core_axis_name="c", subcore_axis_name="s"),
        scratch_types=(
            [pltpu.VMEM((_CH, _D), jnp.float32)] * 6
            + [pltpu.SemaphoreType.DMA] * 10
        ),
    )(x, pos_table)
